# baseline (device time: 20900 ns/iter reference)
import math

import jax
import jax.numpy as jnp
from jax import lax
from jax.experimental import pallas as pl
from jax.experimental.pallas import tpu as pltpu

N_DEV = 4


def kernel(q, k, v):
    s_per, d = q.shape
    scale = 1.0 / math.sqrt(d)

    def body(q_ref, k_ref, v_ref, out_ref, kv_ref, send_sems, recv_sems):
        my_pos = lax.axis_index("i")
        left = (my_pos - 1) % N_DEV
        right = (my_pos + 1) % N_DEV

        barrier_sem = pltpu.get_barrier_semaphore()
        for nbr in [left, right]:
            pl.semaphore_signal(
                barrier_sem, inc=1,
                device_id=(nbr,), device_id_type=pl.DeviceIdType.MESH,
            )
        pl.semaphore_wait(barrier_sem, 2)

        kv_ref[0, 0] = k_ref[...].astype(jnp.bfloat16)
        kv_ref[0, 1] = v_ref[...].astype(jnp.bfloat16)

        q_bf = (q_ref[...] * scale).astype(jnp.bfloat16)
        m = jnp.full((s_per, 1), -jnp.inf, dtype=jnp.float32)
        l = jnp.zeros((s_per, 1), dtype=jnp.float32)
        acc = jnp.zeros((s_per, d), dtype=jnp.float32)

        for h in range(N_DEV):
            slot = h % 2
            rdma = None
            if h < N_DEV - 1:
                rdma = pltpu.make_async_remote_copy(
                    src_ref=kv_ref.at[slot],
                    dst_ref=kv_ref.at[(h + 1) % 2],
                    send_sem=send_sems.at[slot],
                    recv_sem=recv_sems.at[(h + 1) % 2],
                    device_id=(right,),
                    device_id_type=pl.DeviceIdType.MESH,
                )
                rdma.start()

            kb = kv_ref[slot, 0]
            vb = kv_ref[slot, 1]
            s = jnp.dot(q_bf, kb.T, preferred_element_type=jnp.float32)
            m_new = jnp.maximum(m, jnp.max(s, axis=1, keepdims=True))
            p = jnp.exp(s - m_new)
            alpha = jnp.exp(m - m_new)
            l = l * alpha + jnp.sum(p, axis=1, keepdims=True)
            acc = acc * alpha + jnp.dot(
                p.astype(jnp.bfloat16), vb, preferred_element_type=jnp.float32
            )
            m = m_new

            if rdma is not None:
                rdma.wait()

        out_ref[...] = acc / l

    return pl.pallas_call(
        body,
        out_shape=jax.ShapeDtypeStruct((s_per, d), jnp.float32),
        in_specs=[pl.BlockSpec(memory_space=pltpu.VMEM)] * 3,
        out_specs=pl.BlockSpec(memory_space=pltpu.VMEM),
        scratch_shapes=[
            pltpu.VMEM((2, 2, s_per, d), jnp.bfloat16),
            pltpu.SemaphoreType.DMA((2,)),
            pltpu.SemaphoreType.DMA((2,)),
        ],
        compiler_params=pltpu.CompilerParams(collective_id=0),
    )(q, k, v)


# device time: 14525 ns/iter; 1.4389x vs baseline; 1.4389x over previous
import math

import jax
import jax.numpy as jnp
from jax import lax
from jax.experimental import pallas as pl
from jax.experimental.pallas import tpu as pltpu

N_DEV = 4


def kernel(q, k, v):
    s_per, d = q.shape
    scale = 1.0 / math.sqrt(d)

    def body(q_ref, k_ref, v_ref, out_ref, kv_ref, send_sems, recv_sems):
        my_pos = lax.axis_index("i")
        peers = [(my_pos + t) % N_DEV for t in range(1, N_DEV)]

        barrier_sem = pltpu.get_barrier_semaphore()
        for nbr in peers:
            pl.semaphore_signal(
                barrier_sem, inc=1,
                device_id=(nbr,), device_id_type=pl.DeviceIdType.MESH,
            )
        pl.semaphore_wait(barrier_sem, N_DEV - 1)

        kv_ref[0, 0] = k_ref[...].astype(jnp.bfloat16)
        kv_ref[0, 1] = v_ref[...].astype(jnp.bfloat16)

        sends = []
        for t in (1, 3, 2):
            rdma = pltpu.make_async_remote_copy(
                src_ref=kv_ref.at[0],
                dst_ref=kv_ref.at[N_DEV - t],
                send_sem=send_sems.at[t],
                recv_sem=recv_sems.at[N_DEV - t],
                device_id=((my_pos + t) % N_DEV,),
                device_id_type=pl.DeviceIdType.MESH,
            )
            rdma.start()
            sends.append(rdma)

        q_bf = (q_ref[...] * scale).astype(jnp.bfloat16)
        m = jnp.full((s_per, 1), -jnp.inf, dtype=jnp.float32)
        l = jnp.zeros((s_per, 1), dtype=jnp.float32)
        acc = jnp.zeros((s_per, d), dtype=jnp.float32)

        for slot in (0, 1, 3, 2):
            if slot != 0:
                recv = pltpu.make_async_remote_copy(
                    src_ref=kv_ref.at[0],
                    dst_ref=kv_ref.at[slot],
                    send_sem=send_sems.at[0],
                    recv_sem=recv_sems.at[slot],
                    device_id=(my_pos,),
                    device_id_type=pl.DeviceIdType.MESH,
                )
                recv.wait_recv()
            kb = kv_ref[slot, 0]
            vb = kv_ref[slot, 1]
            s = jnp.dot(q_bf, kb.T, preferred_element_type=jnp.float32)
            m_new = jnp.maximum(m, jnp.max(s, axis=1, keepdims=True))
            p = jnp.exp(s - m_new)
            alpha = jnp.exp(m - m_new)
            l = l * alpha + jnp.sum(p, axis=1, keepdims=True)
            acc = acc * alpha + jnp.dot(
                p.astype(jnp.bfloat16), vb, preferred_element_type=jnp.float32
            )
            m = m_new

        out_ref[...] = acc / l

        for rdma in sends:
            rdma.wait_send()

    return pl.pallas_call(
        body,
        out_shape=jax.ShapeDtypeStruct((s_per, d), jnp.float32),
        in_specs=[pl.BlockSpec(memory_space=pltpu.VMEM)] * 3,
        out_specs=pl.BlockSpec(memory_space=pltpu.VMEM),
        scratch_shapes=[
            pltpu.VMEM((N_DEV, 2, s_per, d), jnp.bfloat16),
            pltpu.SemaphoreType.DMA((N_DEV,)),
            pltpu.SemaphoreType.DMA((N_DEV,)),
        ],
        compiler_params=pltpu.CompilerParams(collective_id=0),
    )(q, k, v)
